# baseline (speedup 1.0000x reference)
import functools

import jax
import jax.numpy as jnp
from jax import lax
from jax.experimental import pallas as pl
from jax.experimental.pallas import tpu as pltpu
from jax.experimental.pallas import tpu_sc as plsc

B = 16384
K = 32
NC = 2
NS = 16
NW = NC * NS
BPW = B // NW
IDX_CHUNK = 128
NCHUNK = BPW // IDX_CHUNK

_mesh = plsc.VectorSubcoreMesh(core_axis_name="c", subcore_axis_name="s")


@functools.partial(
    pl.kernel,
    mesh=_mesh,
    compiler_params=pltpu.CompilerParams(
        needs_layout_passes=False, use_tc_tiling_on_sc=False),
    out_type=jax.ShapeDtypeStruct((B,), jnp.float32),
    scratch_types=[
        pltpu.VMEM((NCHUNK, IDX_CHUNK), jnp.int32),
        pltpu.VMEM((NCHUNK, IDX_CHUNK), jnp.int32),
        pltpu.VMEM((BPW, K), jnp.float32),
        pltpu.VMEM((BPW, K), jnp.float32),
        pltpu.VMEM((BPW,), jnp.float32),
        pltpu.SemaphoreType.DMA,
    ],
)
def _mf_kernel(uidx_hbm, iidx_hbm, user_hbm, item_hbm, out_hbm,
               uidx_v, iidx_v, u_rows, i_rows, out_v, sem):
    wid = lax.axis_index("s") * NC + lax.axis_index("c")
    base = wid * BPW

    pltpu.sync_copy(uidx_hbm.at[pl.ds(wid * NCHUNK, NCHUNK)], uidx_v)
    pltpu.sync_copy(iidx_hbm.at[pl.ds(wid * NCHUNK, NCHUNK)], iidx_v)

    copies = []
    for j in range(NCHUNK):
        copies.append(pltpu.async_copy(
            user_hbm.at[uidx_v.at[j]],
            u_rows.at[pl.ds(j * IDX_CHUNK, IDX_CHUNK)], sem))
        copies.append(pltpu.async_copy(
            item_hbm.at[iidx_v.at[j]],
            i_rows.at[pl.ds(j * IDX_CHUNK, IDX_CHUNK)], sem))
    for c in copies:
        c.wait()

    lanes = lax.iota(jnp.int32, 16)

    def group_body(g, _):
        rows = lanes + g * 16
        acc0 = jnp.zeros((16,), jnp.float32)
        acc1 = jnp.zeros((16,), jnp.float32)
        for k in range(0, K, 2):
            col0 = jnp.full((16,), k, jnp.int32)
            col1 = jnp.full((16,), k + 1, jnp.int32)
            acc0 = acc0 + (plsc.load_gather(u_rows, [rows, col0])
                           * plsc.load_gather(i_rows, [rows, col0]))
            acc1 = acc1 + (plsc.load_gather(u_rows, [rows, col1])
                           * plsc.load_gather(i_rows, [rows, col1]))
        out_v[pl.ds(g * 16, 16)] = acc0 + acc1
        return 0

    lax.fori_loop(0, BPW // 16, group_body, 0)

    pltpu.sync_copy(out_v, out_hbm.at[pl.ds(base, BPW)])


def kernel(x, user_emb, item_emb):
    uidx = x[:, 0].reshape(B // IDX_CHUNK, IDX_CHUNK).astype(jnp.int32)
    iidx = x[:, 1].reshape(B // IDX_CHUNK, IDX_CHUNK).astype(jnp.int32)
    return _mf_kernel(uidx, iidx, user_emb, item_emb)
